# Initial kernel scaffold; baseline (speedup 1.0000x reference)
import jax
import jax.numpy as jnp
from jax.experimental import pallas as pl


def kernel(x, edge_index, W1, b1, W2, b2):
    raise NotImplementedError("write your pallas kernel here")


# speedup vs baseline: 15.9462x; 15.9462x over previous
import functools

import jax
import jax.numpy as jnp
from jax import lax
from jax.experimental import pallas as pl
from jax.experimental.pallas import tpu as pltpu
from jax.experimental.pallas import tpu_sc as plsc

N = 10000
D = 128
H = 128
C = 2

NP = 10240
K = 128
NCHUNKS = 80
EP = 32 * NCHUNKS * K
NTILES = 16
RPT = NP // NTILES


def _fill(ref, rows, width, value):
    val = jnp.full((16,), value, jnp.float32)

    def body(i, _):
        for j in range(width // 16):
            ref[i, pl.ds(j * 16, 16)] = val
        return 0

    lax.fori_loop(0, rows, body, 0)


def _make_deg_kernel():
    mesh = plsc.VectorSubcoreMesh(core_axis_name="c", subcore_axis_name="s")

    @functools.partial(
        pl.kernel,
        mesh=mesh,
        out_type=jax.ShapeDtypeStruct((2 * NP, 16), jnp.float32),
        compiler_params=pltpu.CompilerParams(use_tc_tiling_on_sc=False),
        scratch_types=[
            pltpu.VMEM((1, K), jnp.int32),
            pltpu.VMEM((K, 16), jnp.float32),
            pltpu.VMEM_SHARED((NP, 16), jnp.float32),
        ],
    )
    def k(dst_hbm, out_hbm, didx, rows, acc):
        cid = lax.axis_index("c")
        sid = lax.axis_index("s")
        wid = sid * 2 + cid
        _fill(rows, K, 16, 0.0)
        for b in range(RPT // K):
            pltpu.sync_copy(rows, acc.at[pl.ds(sid * RPT + b * K, K)])
        _fill(rows, K, 16, 1.0)
        plsc.subcore_barrier()

        ebase = wid * (NCHUNKS * K)

        def chunk(i, _):
            base = ebase + i * K
            pltpu.sync_copy(dst_hbm.at[pl.ds(base, K)], didx.at[0])
            pltpu.sync_copy(rows, acc.at[didx.at[0]], add=True)
            return 0

        lax.fori_loop(0, NCHUNKS, chunk, 0)
        plsc.subcore_barrier()
        pltpu.sync_copy(
            acc.at[pl.ds(sid * RPT, RPT)],
            out_hbm.at[pl.ds(cid * NP + sid * RPT, RPT)],
        )

    return k


def _make_agg_kernel(F):
    mesh = plsc.VectorSubcoreMesh(core_axis_name="c", subcore_axis_name="s")

    @functools.partial(
        pl.kernel,
        mesh=mesh,
        out_type=jax.ShapeDtypeStruct((2 * NP, F), jnp.float32),
        compiler_params=pltpu.CompilerParams(use_tc_tiling_on_sc=False),
        scratch_types=[
            pltpu.VMEM((K,), jnp.int32),
            pltpu.VMEM((1, K), jnp.int32),
            pltpu.VMEM((K, F), jnp.float32),
            pltpu.VMEM_SHARED((NP, F), jnp.float32),
            pltpu.SemaphoreType.DMA,
        ],
    )
    def k(src_hbm, dst_hbm, g_hbm, out_hbm, sidx, didx, rows, acc, sem):
        cid = lax.axis_index("c")
        sid = lax.axis_index("s")
        wid = sid * 2 + cid
        _fill(rows, K, F, 0.0)
        for b in range(RPT // K):
            pltpu.sync_copy(rows, acc.at[pl.ds(sid * RPT + b * K, K)])
        plsc.subcore_barrier()

        ebase = wid * (NCHUNKS * K)

        def chunk(i, _):
            base = ebase + i * K
            pltpu.sync_copy(src_hbm.at[pl.ds(base, K)], sidx)
            pltpu.sync_copy(dst_hbm.at[pl.ds(base, K)], didx.at[0])
            pltpu.async_copy(g_hbm.at[sidx], rows, sem).wait()
            pltpu.sync_copy(rows, acc.at[didx.at[0]], add=True)
            return 0

        lax.fori_loop(0, NCHUNKS, chunk, 0)
        plsc.subcore_barrier()
        pltpu.sync_copy(
            acc.at[pl.ds(sid * RPT, RPT)],
            out_hbm.at[pl.ds(cid * NP + sid * RPT, RPT)],
        )

    return k


_deg_kernel = _make_deg_kernel()
_agg128 = _make_agg_kernel(H)
_agg16 = _make_agg_kernel(16)

BN = 256


def _dinv(d0_ref, d1_ref):
    deg = d0_ref[:, 0:1] + d1_ref[:, 0:1] + 1.0
    return lax.rsqrt(deg)


def _t1_body(x_ref, w_ref, d0_ref, d1_ref, o_ref):
    h = jnp.dot(x_ref[...], w_ref[...], preferred_element_type=jnp.float32)
    o_ref[...] = h * _dinv(d0_ref, d1_ref)


def _t2_body(u0_ref, u1_ref, g_ref, d0_ref, d1_ref, b1_ref, w2_ref, o_ref):
    dinv = _dinv(d0_ref, d1_ref)
    u = u0_ref[...] + u1_ref[...] + g_ref[...]
    h1 = jnp.maximum(u * dinv + b1_ref[...], 0.0)
    z = jnp.dot(h1, w2_ref[...], preferred_element_type=jnp.float32)
    o_ref[...] = z * dinv


def _t3_body(u0_ref, u1_ref, g_ref, d0_ref, d1_ref, b2_ref, o_ref):
    dinv = _dinv(d0_ref, d1_ref)
    z = (u0_ref[...] + u1_ref[...] + g_ref[...]) * dinv + b2_ref[...]
    m = jnp.maximum(z[:, 0:1], z[:, 1:2])
    ez = jnp.exp(z - m)
    s = ez[:, 0:1] + ez[:, 1:2]
    o_ref[...] = ez / s


def _row_spec(f):
    return pl.BlockSpec((BN, f), lambda i: (i, 0))


def _full_spec(r, f):
    return pl.BlockSpec((r, f), lambda i: (0, 0))


_GRID = (NP // BN,)

_t1 = pl.pallas_call(
    _t1_body,
    grid=_GRID,
    in_specs=[_row_spec(D), _full_spec(D, H), _row_spec(16), _row_spec(16)],
    out_specs=_row_spec(H),
    out_shape=jax.ShapeDtypeStruct((NP, H), jnp.float32),
)

_t2 = pl.pallas_call(
    _t2_body,
    grid=_GRID,
    in_specs=[
        _row_spec(H),
        _row_spec(H),
        _row_spec(H),
        _row_spec(16),
        _row_spec(16),
        _full_spec(1, H),
        _full_spec(H, 16),
    ],
    out_specs=_row_spec(16),
    out_shape=jax.ShapeDtypeStruct((NP, 16), jnp.float32),
)

_t3 = pl.pallas_call(
    _t3_body,
    grid=_GRID,
    in_specs=[
        _row_spec(16),
        _row_spec(16),
        _row_spec(16),
        _row_spec(16),
        _row_spec(16),
        _full_spec(1, 16),
    ],
    out_specs=_row_spec(16),
    out_shape=jax.ShapeDtypeStruct((NP, 16), jnp.float32),
)


def kernel(x, edge_index, W1, b1, W2, b2):
    ei = edge_index.astype(jnp.int32)
    E = ei.shape[1]
    npad = EP - E
    padidx = N + (jnp.arange(npad, dtype=jnp.int32) % (NP - N))
    src = jnp.concatenate([ei[0], padidx])
    dst = jnp.concatenate([ei[1], padidx])
    xp = jnp.concatenate([x, jnp.zeros((NP - N, D), x.dtype)])
    w2p = jnp.pad(W2, ((0, 0), (0, 16 - C)))
    b1r = b1.reshape(1, H)
    b2r = jnp.pad(b2, (0, 16 - C)).reshape(1, 16)

    dp = _deg_kernel(dst)
    d0, d1 = dp[:NP], dp[NP:]
    g1 = _t1(xp, W1, d0, d1)
    u1 = _agg128(src, dst, g1)
    g2 = _t2(u1[:NP], u1[NP:], g1, d0, d1, b1r, w2p)
    u2 = _agg16(src, dst, g2)
    outp = _t3(u2[:NP], u2[NP:], g2, d0, d1, b2r)
    return outp[:N, :C]


# speedup vs baseline: 24.9424x; 1.5642x over previous
import functools

import jax
import jax.numpy as jnp
from jax import lax
from jax.experimental import pallas as pl
from jax.experimental.pallas import tpu as pltpu
from jax.experimental.pallas import tpu_sc as plsc

N = 10000
D = 128
H = 128
C = 2

NP = 10240
K = 128
NCHUNKS = 80
EP = 32 * NCHUNKS * K
NTILES = 16
RPT = NP // NTILES
RING = 2
EPT = NCHUNKS * K


def _fill(ref, rows, width, value):
    val = jnp.full((16,), value, jnp.float32)

    def body(i, _):
        for j in range(width // 16):
            ref[i, pl.ds(j * 16, 16)] = val
        return 0

    lax.fori_loop(0, rows, body, 0)


def _zero_acc_slab(buf, acc, sid, width):
    _fill(buf, K, width, 0.0)
    for b in range(RPT // K):
        pltpu.sync_copy(buf, acc.at[pl.ds(sid * RPT + b * K, K)])


def _make_deg_kernel():
    mesh = plsc.VectorSubcoreMesh(core_axis_name="c", subcore_axis_name="s")

    @functools.partial(
        pl.kernel,
        mesh=mesh,
        out_type=jax.ShapeDtypeStruct((2 * NP, 16), jnp.float32),
        compiler_params=pltpu.CompilerParams(use_tc_tiling_on_sc=False),
        scratch_types=[
            pltpu.VMEM((NCHUNKS, K), jnp.int32),
            pltpu.VMEM((K, 16), jnp.float32),
            pltpu.VMEM_SHARED((NP, 16), jnp.float32),
            pltpu.SemaphoreType.DMA,
        ],
    )
    def k(dst_hbm, out_hbm, didx, ones, acc, sem):
        cid = lax.axis_index("c")
        sid = lax.axis_index("s")
        wid = sid * 2 + cid
        _zero_acc_slab(ones, acc, sid, 16)
        _fill(ones, K, 16, 1.0)
        pltpu.sync_copy(dst_hbm.at[pl.ds(wid * NCHUNKS, NCHUNKS)], didx)
        plsc.subcore_barrier()
        descs = [
            pltpu.async_copy(ones, acc.at[didx.at[c]], sem, add=True)
            for c in range(NCHUNKS)
        ]
        for d in descs:
            d.wait()
        plsc.subcore_barrier()
        pltpu.sync_copy(
            acc.at[pl.ds(sid * RPT, RPT)],
            out_hbm.at[pl.ds(cid * NP + sid * RPT, RPT)],
        )

    return k


def _make_agg_kernel(F):
    mesh = plsc.VectorSubcoreMesh(core_axis_name="c", subcore_axis_name="s")
    B = 2 if F > 16 else 4
    DR = 2 * B

    @functools.partial(
        pl.kernel,
        mesh=mesh,
        out_type=jax.ShapeDtypeStruct((2 * NP, F), jnp.float32),
        compiler_params=pltpu.CompilerParams(use_tc_tiling_on_sc=False),
        scratch_types=[
            pltpu.VMEM((B, K), jnp.int32),
            pltpu.VMEM((DR, K), jnp.int32),
            pltpu.VMEM((B, K, F), jnp.float32),
            pltpu.VMEM_SHARED((NP, F), jnp.float32),
        ]
        + [pltpu.SemaphoreType.DMA] * (2 * B + DR + B),
    )
    def k(src_hbm, dst_hbm, g_hbm, out_hbm, sidx, didx, bufs, acc, *sems):
        isems = sems[:B]
        jsems = sems[B:B + DR]
        gsems = sems[B + DR:2 * B + DR]
        ssems = sems[2 * B + DR:]
        cid = lax.axis_index("c")
        sid = lax.axis_index("s")
        wid = sid * 2 + cid
        _zero_acc_slab(bufs.at[0], acc, sid, F)
        plsc.subcore_barrier()
        ebase = wid * EPT

        def load_sidx(c):
            b = c % B
            return pltpu.async_copy(
                src_hbm.at[pl.ds(ebase + c * K, K)], sidx.at[b], isems[b])

        def load_didx(c):
            d = c % DR
            return pltpu.async_copy(
                dst_hbm.at[pl.ds(ebase + c * K, K)], didx.at[d], jsems[d])

        def gather(c):
            b = c % B
            return pltpu.async_copy(
                g_hbm.at[sidx.at[b]], bufs.at[b], gsems[b])

        def scatter(c):
            b = c % B
            return pltpu.async_copy(
                bufs.at[b], acc.at[didx.at[c % DR]], ssems[b], add=True)

        sdx = {}
        ddx = {}
        gd = {}
        sd = {}
        for c in range(min(B, NCHUNKS)):
            sdx[c] = load_sidx(c)
            ddx[c] = load_didx(c)
        for c in range(NCHUNKS):
            sdx[c].wait()
            ddx[c].wait()
            if c >= B:
                sd[c - B].wait()
            if c + B < NCHUNKS:
                ddx[c + B] = load_didx(c + B)
            gd[c] = gather(c)
            gd[c].wait()
            if c + B < NCHUNKS:
                sdx[c + B] = load_sidx(c + B)
            sd[c] = scatter(c)
        for c in range(max(0, NCHUNKS - B), NCHUNKS):
            sd[c].wait()
        plsc.subcore_barrier()
        pltpu.sync_copy(
            acc.at[pl.ds(sid * RPT, RPT)],
            out_hbm.at[pl.ds(cid * NP + sid * RPT, RPT)],
        )

    return k


_deg_kernel = _make_deg_kernel()
_agg128 = _make_agg_kernel(H)
_agg16 = _make_agg_kernel(16)

BN = 256


def _dinv(d0_ref, d1_ref):
    deg = d0_ref[:, 0:1] + d1_ref[:, 0:1] + 1.0
    return lax.rsqrt(deg)


def _t1a_body(x_ref, w_ref, o_ref):
    o_ref[...] = jnp.dot(x_ref[...], w_ref[...],
                         preferred_element_type=jnp.float32)


def _t1b_body(h_ref, d0_ref, d1_ref, o_ref):
    o_ref[...] = h_ref[...] * _dinv(d0_ref, d1_ref)


def _t2_body(u0_ref, u1_ref, g_ref, d0_ref, d1_ref, b1_ref, w2_ref, o_ref):
    dinv = _dinv(d0_ref, d1_ref)
    u = u0_ref[...] + u1_ref[...] + g_ref[...]
    h1 = jnp.maximum(u * dinv + b1_ref[...], 0.0)
    z = jnp.dot(h1, w2_ref[...], preferred_element_type=jnp.float32)
    o_ref[...] = z * dinv


def _t3_body(u0_ref, u1_ref, g_ref, d0_ref, d1_ref, b2_ref, o_ref):
    dinv = _dinv(d0_ref, d1_ref)
    z = (u0_ref[...] + u1_ref[...] + g_ref[...]) * dinv + b2_ref[...]
    m = jnp.maximum(z[:, 0:1], z[:, 1:2])
    ez = jnp.exp(z - m)
    s = ez[:, 0:1] + ez[:, 1:2]
    o_ref[...] = ez / s


def _row_spec(f):
    return pl.BlockSpec((BN, f), lambda i: (i, 0))


def _full_spec(r, f):
    return pl.BlockSpec((r, f), lambda i: (0, 0))


_GRID = (NP // BN,)

_t1a = pl.pallas_call(
    _t1a_body,
    grid=_GRID,
    in_specs=[_row_spec(D), _full_spec(D, H)],
    out_specs=_row_spec(H),
    out_shape=jax.ShapeDtypeStruct((NP, H), jnp.float32),
)

_t1b = pl.pallas_call(
    _t1b_body,
    grid=_GRID,
    in_specs=[_row_spec(H), _row_spec(16), _row_spec(16)],
    out_specs=_row_spec(H),
    out_shape=jax.ShapeDtypeStruct((NP, H), jnp.float32),
)

_t2 = pl.pallas_call(
    _t2_body,
    grid=_GRID,
    in_specs=[
        _row_spec(H),
        _row_spec(H),
        _row_spec(H),
        _row_spec(16),
        _row_spec(16),
        _full_spec(1, H),
        _full_spec(H, 16),
    ],
    out_specs=_row_spec(16),
    out_shape=jax.ShapeDtypeStruct((NP, 16), jnp.float32),
)

_t3 = pl.pallas_call(
    _t3_body,
    grid=_GRID,
    in_specs=[
        _row_spec(16),
        _row_spec(16),
        _row_spec(16),
        _row_spec(16),
        _row_spec(16),
        _full_spec(1, 16),
    ],
    out_specs=_row_spec(16),
    out_shape=jax.ShapeDtypeStruct((NP, 16), jnp.float32),
)


def kernel(x, edge_index, W1, b1, W2, b2):
    ei = edge_index.astype(jnp.int32)
    E = ei.shape[1]
    npad = EP - E
    padidx = N + (jnp.arange(npad, dtype=jnp.int32) % (NP - N))
    src = jnp.concatenate([ei[0], padidx])
    dst = jnp.concatenate([ei[1], padidx])
    dst2d = dst.reshape(EP // K, K)
    xp = jnp.concatenate([x, jnp.zeros((NP - N, D), x.dtype)])
    w2p = jnp.pad(W2, ((0, 0), (0, 16 - C)))
    b1r = b1.reshape(1, H)
    b2r = jnp.pad(b2, (0, 16 - C)).reshape(1, 16)

    dp = _deg_kernel(dst2d)
    d0, d1 = dp[:NP], dp[NP:]
    h = _t1a(xp, W1)
    g1 = _t1b(h, d0, d1)
    u1 = _agg128(src, dst, g1)
    g2 = _t2(u1[:NP], u1[NP:], g1, d0, d1, b1r, w2p)
    u2 = _agg16(src, dst, g2)
    outp = _t3(u2[:NP], u2[NP:], g2, d0, d1, b2r)
    return outp[:N, :C]


# speedup vs baseline: 35.9378x; 1.4408x over previous
import functools

import jax
import jax.numpy as jnp
from jax import lax
from jax.experimental import pallas as pl
from jax.experimental.pallas import tpu as pltpu
from jax.experimental.pallas import tpu_sc as plsc

N = 10000
D = 128
H = 128
C = 2
E = 320000

K = 128
CH = E // K
NTILES = 16
NW = 32
TPW = CH // NW
EX = CH - NW * TPW
RPT = N // NTILES


def _fill(ref, rows, width, value):
    val = jnp.full((16,), value, jnp.float32)

    def body(i, _):
        for j in range(width // 16):
            ref[i, pl.ds(j * 16, 16)] = val
        return 0

    lax.fori_loop(0, rows, body, 0)


def _zero_acc_slab(buf, acc, sid, width):
    _fill(buf, K, width, 0.0)
    for b in range(5):
        pltpu.sync_copy(
            buf.at[pl.ds(0, RPT // 5)],
            acc.at[pl.ds(sid * RPT + b * (RPT // 5), RPT // 5)],
        )


def _make_deg_kernel():
    mesh = plsc.VectorSubcoreMesh(core_axis_name="c", subcore_axis_name="s")

    @functools.partial(
        pl.kernel,
        mesh=mesh,
        out_type=jax.ShapeDtypeStruct((2 * N, 16), jnp.float32),
        compiler_params=pltpu.CompilerParams(use_tc_tiling_on_sc=False),
        scratch_types=[
            pltpu.VMEM((TPW + 1, K), jnp.int32),
            pltpu.VMEM((K, 16), jnp.float32),
            pltpu.VMEM_SHARED((N, 16), jnp.float32),
            pltpu.SemaphoreType.DMA,
        ],
    )
    def k(dst_hbm, out_hbm, didx, ones, acc, sem):
        cid = lax.axis_index("c")
        sid = lax.axis_index("s")
        wid = sid * 2 + cid
        has_tail = wid < EX
        _zero_acc_slab(ones, acc, sid, 16)
        _fill(ones, K, 16, 1.0)
        pltpu.sync_copy(dst_hbm.at[pl.ds(wid * TPW, TPW)],
                        didx.at[pl.ds(0, TPW)])

        @pl.when(has_tail)
        def _():
            pltpu.sync_copy(dst_hbm.at[pl.ds(NW * TPW + wid, 1)],
                            didx.at[pl.ds(TPW, 1)])

        plsc.subcore_barrier()
        descs = [
            pltpu.async_copy(ones, acc.at[didx.at[c]], sem, add=True)
            for c in range(TPW)
        ]

        @pl.when(has_tail)
        def _():
            pltpu.make_async_copy(
                ones, acc.at[didx.at[TPW]], sem).start(add=True)

        for d in descs:
            d.wait()

        @pl.when(has_tail)
        def _():
            pltpu.make_async_copy(ones, acc.at[didx.at[TPW]], sem).wait()

        plsc.subcore_barrier()
        pltpu.sync_copy(
            acc.at[pl.ds(sid * RPT, RPT)],
            out_hbm.at[pl.ds(cid * N + sid * RPT, RPT)],
        )

    return k


def _make_agg_kernel(F, B):
    mesh = plsc.VectorSubcoreMesh(core_axis_name="c", subcore_axis_name="s")
    L = B - 1
    DR = B + 1
    NCHL = TPW + 1

    @functools.partial(
        pl.kernel,
        mesh=mesh,
        out_type=jax.ShapeDtypeStruct((2 * N, F), jnp.float32),
        compiler_params=pltpu.CompilerParams(use_tc_tiling_on_sc=False),
        scratch_types=[
            pltpu.VMEM((B, K), jnp.int32),
            pltpu.VMEM((DR, K), jnp.int32),
            pltpu.VMEM((B, K, F), jnp.float32),
            pltpu.VMEM_SHARED((N, F), jnp.float32),
        ]
        + [pltpu.SemaphoreType.DMA] * (3 * B + DR),
    )
    def k(src_hbm, dst_hbm, g_hbm, out_hbm, sidx, didx, bufs, acc, *sems):
        isems = sems[:B]
        jsems = sems[B:B + DR]
        gsems = sems[B + DR:2 * B + DR]
        ssems = sems[2 * B + DR:]
        cid = lax.axis_index("c")
        sid = lax.axis_index("s")
        wid = sid * 2 + cid
        has_tail = wid < EX
        _zero_acc_slab(bufs.at[0], acc, sid, F)
        plsc.subcore_barrier()
        ebase = wid * (TPW * K)

        def off(m):
            if m < TPW:
                return ebase + m * K
            return NW * TPW * K + wid * K

        def guarded(m, fn):
            if m < TPW:
                fn()
            else:
                pl.when(has_tail)(fn)

        def sidx_desc(m):
            b = m % B
            return pltpu.make_async_copy(
                src_hbm.at[pl.ds(off(m), K)], sidx.at[b], isems[b])

        def didx_desc(m):
            d = m % DR
            return pltpu.make_async_copy(
                dst_hbm.at[pl.ds(off(m), K)], didx.at[d], jsems[d])

        def gather_desc(m):
            b = m % B
            return pltpu.make_async_copy(
                g_hbm.at[sidx.at[b]], bufs.at[b], gsems[b])

        def scatter_start(m):
            b = m % B
            pltpu.make_async_copy(
                bufs.at[b], acc.at[didx.at[m % DR]], ssems[b]).start(add=True)

        def scatter_wait(m):
            b = m % B
            pltpu.make_async_copy(
                bufs.at[b], acc.at[didx.at[m % DR]], ssems[b]).wait()

        for m in range(min(B, NCHL)):
            guarded(m, lambda m=m: sidx_desc(m).start())
        for m in range(min(B, NCHL)):
            guarded(m, lambda m=m: didx_desc(m).start())
        for m in range(min(L, NCHL)):
            guarded(m, lambda m=m: sidx_desc(m).wait())
            guarded(m, lambda m=m: gather_desc(m).start())

        for c in range(NCHL):
            guarded(c, lambda c=c: gather_desc(c).wait())
            if c >= 1:
                guarded(c - 1, lambda c=c: scatter_wait(c - 1))
            if c + L < NCHL:
                guarded(c + L, lambda c=c: sidx_desc(c + L).wait())
                guarded(c + L, lambda c=c: gather_desc(c + L).start())
            if c + B < NCHL:
                guarded(c + B, lambda c=c: sidx_desc(c + B).start())
            guarded(c, lambda c=c: didx_desc(c).wait())
            guarded(c, lambda c=c: scatter_start(c))
            if c + B < NCHL:
                guarded(c + B, lambda c=c: didx_desc(c + B).start())
        guarded(NCHL - 1, lambda: scatter_wait(NCHL - 1))
        plsc.subcore_barrier()
        pltpu.sync_copy(
            acc.at[pl.ds(sid * RPT, RPT)],
            out_hbm.at[pl.ds(cid * N + sid * RPT, RPT)],
        )

    return k


_deg_kernel = _make_deg_kernel()
_agg128 = _make_agg_kernel(H, 2)
_agg16 = _make_agg_kernel(16, 4)

BN = 2000


def _dinv(d0, d1):
    return lax.rsqrt(d0[:, 0:1] + d1[:, 0:1] + 1.0)


def _t1a_body(x_ref, w_ref, o_ref):
    o_ref[...] = jnp.dot(x_ref[...], w_ref[...],
                         preferred_element_type=jnp.float32)


def _t1b_body(h_ref, d0_ref, d1_ref, o_ref):
    o_ref[...] = h_ref[...] * _dinv(d0_ref[...], d1_ref[...])


def _t2_body(u0_ref, u1_ref, g_ref, d0_ref, d1_ref, b1_ref, w2_ref, o_ref):
    dinv = _dinv(d0_ref[...], d1_ref[...])
    u = u0_ref[...] + u1_ref[...] + g_ref[...]
    h1 = jnp.maximum(u * dinv + b1_ref[...], 0.0)
    z = jnp.dot(h1, w2_ref[...], preferred_element_type=jnp.float32)
    o_ref[...] = z * dinv


def _t3_body(u0_ref, u1_ref, g_ref, d0_ref, d1_ref, b2_ref, o_ref):
    dinv = _dinv(d0_ref[...], d1_ref[...])
    z = (u0_ref[...] + u1_ref[...] + g_ref[...]) * dinv + b2_ref[...]
    m = jnp.maximum(z[:, 0:1], z[:, 1:2])
    ez = jnp.exp(z - m)
    s = ez[:, 0:1] + ez[:, 1:2]
    o_ref[...] = ez / s


def _spec(f, half=None):
    if half is None:
        return pl.BlockSpec((BN, f), lambda i: (i, 0))
    o = half * (N // BN)
    return pl.BlockSpec((BN, f), lambda i, o=o: (i + o, 0))


def _const_spec(r, f):
    return pl.BlockSpec((r, f), lambda i: (0, 0))


_GRID = (N // BN,)

_t1a = pl.pallas_call(
    _t1a_body,
    grid=_GRID,
    in_specs=[_spec(D), _const_spec(D, H)],
    out_specs=_spec(H),
    out_shape=jax.ShapeDtypeStruct((N, H), jnp.float32),
)

_t1b = pl.pallas_call(
    _t1b_body,
    grid=_GRID,
    in_specs=[_spec(H), _spec(16, 0), _spec(16, 1)],
    out_specs=_spec(H),
    out_shape=jax.ShapeDtypeStruct((N, H), jnp.float32),
)

_t2 = pl.pallas_call(
    _t2_body,
    grid=_GRID,
    in_specs=[
        _spec(H, 0),
        _spec(H, 1),
        _spec(H),
        _spec(16, 0),
        _spec(16, 1),
        _const_spec(1, H),
        _const_spec(H, 16),
    ],
    out_specs=_spec(16),
    out_shape=jax.ShapeDtypeStruct((N, 16), jnp.float32),
)

_t3 = pl.pallas_call(
    _t3_body,
    grid=_GRID,
    in_specs=[
        _spec(16, 0),
        _spec(16, 1),
        _spec(16),
        _spec(16, 0),
        _spec(16, 1),
        _const_spec(1, 16),
    ],
    out_specs=_spec(16),
    out_shape=jax.ShapeDtypeStruct((N, 16), jnp.float32),
)


def kernel(x, edge_index, W1, b1, W2, b2):
    ei = edge_index.astype(jnp.int32)
    src = ei[0]
    dst = ei[1]
    dst2d = dst.reshape(CH, K)
    w2p = jnp.pad(W2, ((0, 0), (0, 16 - C)))
    b1r = b1.reshape(1, H)
    b2r = jnp.pad(b2, (0, 16 - C)).reshape(1, 16)

    dp = _deg_kernel(dst2d)
    h = _t1a(x, W1)
    g1 = _t1b(h, dp, dp)
    u1 = _agg128(src, dst, g1)
    g2 = _t2(u1, u1, g1, dp, dp, b1r, w2p)
    u2 = _agg16(src, dst, g2)
    outp = _t3(u2, u2, g2, dp, dp, b2r)
    return outp[:, :C]


# speedup vs baseline: 37.2106x; 1.0354x over previous
import functools

import jax
import jax.numpy as jnp
from jax import lax
from jax.experimental import pallas as pl
from jax.experimental.pallas import tpu as pltpu
from jax.experimental.pallas import tpu_sc as plsc

N = 10000
D = 128
H = 128
C = 2
E = 320000

K = 128
CH = E // K
NTILES = 16
NW = 32
TPW = CH // NW
EX = CH - NW * TPW
RPT = N // NTILES


def _fill(ref, rows, width, value):
    val = jnp.full((16,), value, jnp.float32)

    def body(i, _):
        for j in range(width // 16):
            ref[i, pl.ds(j * 16, 16)] = val
        return 0

    lax.fori_loop(0, rows, body, 0)


def _zero_acc_slab(buf, acc, sid, width):
    _fill(buf, K, width, 0.0)
    for b in range(5):
        pltpu.sync_copy(
            buf.at[pl.ds(0, RPT // 5)],
            acc.at[pl.ds(sid * RPT + b * (RPT // 5), RPT // 5)],
        )


def _make_deg_kernel():
    mesh = plsc.VectorSubcoreMesh(core_axis_name="c", subcore_axis_name="s")

    @functools.partial(
        pl.kernel,
        mesh=mesh,
        out_type=jax.ShapeDtypeStruct((2 * N, 16), jnp.float32),
        compiler_params=pltpu.CompilerParams(use_tc_tiling_on_sc=False),
        scratch_types=[
            pltpu.VMEM((TPW + 1, K), jnp.int32),
            pltpu.VMEM((K, 16), jnp.float32),
            pltpu.VMEM_SHARED((N, 16), jnp.float32),
            pltpu.SemaphoreType.DMA,
            pltpu.SemaphoreType.DMA,
        ],
    )
    def k(ei_hbm, out_hbm, didx, ones, acc, isem, sem):
        cid = lax.axis_index("c")
        sid = lax.axis_index("s")
        wid = sid * 2 + cid
        has_tail = wid < EX
        _zero_acc_slab(ones, acc, sid, 16)
        _fill(ones, K, 16, 1.0)
        ldescs = [
            pltpu.async_copy(ei_hbm.at[wid * TPW + c, 1], didx.at[c], isem)
            for c in range(TPW)
        ]

        @pl.when(has_tail)
        def _():
            pltpu.make_async_copy(
                ei_hbm.at[NW * TPW + wid, 1], didx.at[TPW], isem).start()

        for d in ldescs:
            d.wait()

        @pl.when(has_tail)
        def _():
            pltpu.make_async_copy(
                ei_hbm.at[NW * TPW + wid, 1], didx.at[TPW], isem).wait()

        plsc.subcore_barrier()
        descs = [
            pltpu.async_copy(ones, acc.at[didx.at[c]], sem, add=True)
            for c in range(TPW)
        ]

        @pl.when(has_tail)
        def _():
            pltpu.make_async_copy(
                ones, acc.at[didx.at[TPW]], sem).start(add=True)

        for d in descs:
            d.wait()

        @pl.when(has_tail)
        def _():
            pltpu.make_async_copy(ones, acc.at[didx.at[TPW]], sem).wait()

        plsc.subcore_barrier()
        pltpu.sync_copy(
            acc.at[pl.ds(sid * RPT, RPT)],
            out_hbm.at[pl.ds(cid * N + sid * RPT, RPT)],
        )

    return k


def _make_agg_kernel(F, B):
    mesh = plsc.VectorSubcoreMesh(core_axis_name="c", subcore_axis_name="s")
    L = B - 1
    DR = B + 1
    NCHL = TPW + 1

    @functools.partial(
        pl.kernel,
        mesh=mesh,
        out_type=jax.ShapeDtypeStruct((2 * N, F), jnp.float32),
        compiler_params=pltpu.CompilerParams(use_tc_tiling_on_sc=False),
        scratch_types=[
            pltpu.VMEM((B, K), jnp.int32),
            pltpu.VMEM((DR, K), jnp.int32),
            pltpu.VMEM((B, K, F), jnp.float32),
            pltpu.VMEM_SHARED((N, F), jnp.float32),
        ]
        + [pltpu.SemaphoreType.DMA] * (3 * B + DR),
    )
    def k(ei_hbm, g_hbm, out_hbm, sidx, didx, bufs, acc, *sems):
        isems = sems[:B]
        jsems = sems[B:B + DR]
        gsems = sems[B + DR:2 * B + DR]
        ssems = sems[2 * B + DR:]
        cid = lax.axis_index("c")
        sid = lax.axis_index("s")
        wid = sid * 2 + cid
        has_tail = wid < EX
        _zero_acc_slab(bufs.at[0], acc, sid, F)
        plsc.subcore_barrier()

        def chunk_row(m):
            if m < TPW:
                return wid * TPW + m
            return NW * TPW + wid

        def guarded(m, fn):
            if m < TPW:
                fn()
            else:
                pl.when(has_tail)(fn)

        def sidx_desc(m):
            b = m % B
            return pltpu.make_async_copy(
                ei_hbm.at[chunk_row(m), 0], sidx.at[b], isems[b])

        def didx_desc(m):
            d = m % DR
            return pltpu.make_async_copy(
                ei_hbm.at[chunk_row(m), 1], didx.at[d], jsems[d])

        def gather_desc(m):
            b = m % B
            return pltpu.make_async_copy(
                g_hbm.at[sidx.at[b]], bufs.at[b], gsems[b])

        def scatter_start(m):
            b = m % B
            pltpu.make_async_copy(
                bufs.at[b], acc.at[didx.at[m % DR]], ssems[b]).start(add=True)

        def scatter_wait(m):
            b = m % B
            pltpu.make_async_copy(
                bufs.at[b], acc.at[didx.at[m % DR]], ssems[b]).wait()

        for m in range(min(B, NCHL)):
            guarded(m, lambda m=m: sidx_desc(m).start())
        for m in range(min(B, NCHL)):
            guarded(m, lambda m=m: didx_desc(m).start())
        for m in range(min(L, NCHL)):
            guarded(m, lambda m=m: sidx_desc(m).wait())
            guarded(m, lambda m=m: gather_desc(m).start())

        for c in range(NCHL):
            guarded(c, lambda c=c: gather_desc(c).wait())
            if c >= 1:
                guarded(c - 1, lambda c=c: scatter_wait(c - 1))
            if c + L < NCHL:
                guarded(c + L, lambda c=c: sidx_desc(c + L).wait())
                guarded(c + L, lambda c=c: gather_desc(c + L).start())
            if c + B < NCHL:
                guarded(c + B, lambda c=c: sidx_desc(c + B).start())
            guarded(c, lambda c=c: didx_desc(c).wait())
            guarded(c, lambda c=c: scatter_start(c))
            if c + B < NCHL:
                guarded(c + B, lambda c=c: didx_desc(c + B).start())
        guarded(NCHL - 1, lambda: scatter_wait(NCHL - 1))
        plsc.subcore_barrier()
        pltpu.sync_copy(
            acc.at[pl.ds(sid * RPT, RPT)],
            out_hbm.at[pl.ds(cid * N + sid * RPT, RPT)],
        )

    return k


_deg_kernel = _make_deg_kernel()
_agg128 = _make_agg_kernel(H, 2)
_agg16 = _make_agg_kernel(16, 4)

BN = 2000


def _dinv(d0, d1):
    return lax.rsqrt(d0[:, 0:1] + d1[:, 0:1] + 1.0)


def _t1a_body(x_ref, w_ref, o_ref):
    o_ref[...] = jnp.dot(x_ref[...], w_ref[...],
                         preferred_element_type=jnp.float32)


def _t1b_body(h_ref, d0_ref, d1_ref, o_ref):
    o_ref[...] = h_ref[...] * _dinv(d0_ref[...], d1_ref[...])


def _t2_body(u0_ref, u1_ref, g_ref, d0_ref, d1_ref, b1_ref, w2_ref, o_ref):
    dinv = _dinv(d0_ref[...], d1_ref[...])
    u = u0_ref[...] + u1_ref[...] + g_ref[...]
    h1 = jnp.maximum(u * dinv + b1_ref[...], 0.0)
    z = jnp.dot(h1, w2_ref[...], preferred_element_type=jnp.float32)
    o_ref[...] = z * dinv


def _t3_body(u0_ref, u1_ref, g_ref, d0_ref, d1_ref, b2_ref, o_ref):
    dinv = _dinv(d0_ref[...], d1_ref[...])
    z = (u0_ref[...] + u1_ref[...] + g_ref[...]) * dinv + b2_ref[...]
    m = jnp.maximum(z[:, 0:1], z[:, 1:2])
    ez = jnp.exp(z - m)
    s = ez[:, 0:1] + ez[:, 1:2]
    o_ref[...] = ez / s


def _spec(f, half=None):
    if half is None:
        return pl.BlockSpec((BN, f), lambda i: (i, 0))
    o = half * (N // BN)
    return pl.BlockSpec((BN, f), lambda i, o=o: (i + o, 0))


def _const_spec(r, f):
    return pl.BlockSpec((r, f), lambda i: (0, 0))


_GRID = (N // BN,)

_t1a = pl.pallas_call(
    _t1a_body,
    grid=_GRID,
    in_specs=[_spec(D), _const_spec(D, H)],
    out_specs=_spec(H),
    out_shape=jax.ShapeDtypeStruct((N, H), jnp.float32),
)

_t1b = pl.pallas_call(
    _t1b_body,
    grid=_GRID,
    in_specs=[_spec(H), _spec(16, 0), _spec(16, 1)],
    out_specs=_spec(H),
    out_shape=jax.ShapeDtypeStruct((N, H), jnp.float32),
)

_t2 = pl.pallas_call(
    _t2_body,
    grid=_GRID,
    in_specs=[
        _spec(H, 0),
        _spec(H, 1),
        _spec(H),
        _spec(16, 0),
        _spec(16, 1),
        _const_spec(1, H),
        _const_spec(H, 16),
    ],
    out_specs=_spec(16),
    out_shape=jax.ShapeDtypeStruct((N, 16), jnp.float32),
)

_t3 = pl.pallas_call(
    _t3_body,
    grid=_GRID,
    in_specs=[
        _spec(16, 0),
        _spec(16, 1),
        _spec(16),
        _spec(16, 0),
        _spec(16, 1),
        _const_spec(1, 16),
    ],
    out_specs=_spec(16),
    out_shape=jax.ShapeDtypeStruct((N, 16), jnp.float32),
)


def kernel(x, edge_index, W1, b1, W2, b2):
    ei3 = (edge_index.astype(jnp.int32)
           .reshape(2, CH, K).transpose(1, 0, 2))
    w2p = jnp.pad(W2, ((0, 0), (0, 16 - C)))
    b1r = b1.reshape(1, H)
    b2r = jnp.pad(b2, (0, 16 - C)).reshape(1, 16)

    dp = _deg_kernel(ei3)
    h = _t1a(x, W1)
    g1 = _t1b(h, dp, dp)
    u1 = _agg128(ei3, g1)
    g2 = _t2(u1, u1, g1, dp, dp, b1r, w2p)
    u2 = _agg16(ei3, g2)
    outp = _t3(u2, u2, g2, dp, dp, b2r)
    return outp[:, :C]
